# baseline (speedup 1.0000x reference)
import functools

import jax
import jax.numpy as jnp
from jax import lax
from jax.experimental import pallas as pl
from jax.experimental.pallas import tpu as pltpu
from jax.experimental.pallas import tpu_sc as plsc

N = 10000
D = 128
NPAD = 10112
HALF = NPAD // 2
RSLAB = 316
SLAB = 320
NC = 2
NS = 16
NW = NC * NS
CH = 24
NB = 2
BLK = 632
ACC_R = NS * SLAB
REPC = 64


def _zero_rows(buf, nrows, width):
    def body(r, _):
        for j in range(width // 16):
            buf[r, pl.ds(j * 16, 16)] = jnp.zeros((16,), jnp.float32)
        return 0
    lax.fori_loop(0, nrows, body, 0)


def _sc_scatter_body(table_hbm, src_hbm, dst_hbm, out_hbm,
                     sb0, sb1, db0, db1, b0, b1, table_sp, acc_sp,
                     gs0, gs1, ssem, isem, nch, tab_rows):
    bufs = (b0, b1)
    gsems = (gs0, gs1)
    sbufs = (sb0, sb1)
    dbufs = (db0, db1)

    cid = lax.axis_index("c")
    sid = lax.axis_index("s")

    tpt = tab_rows // NS
    pltpu.sync_copy(table_hbm.at[pl.ds(sid * tpt, tpt)],
                    table_sp.at[pl.ds(sid * tpt, tpt)])

    _zero_rows(bufs[0], CH, D)
    zb = sid * SLAB
    for k in range(SLAB // CH):
        pltpu.sync_copy(bufs[0], acc_sp.at[pl.ds(zb + k * CH, CH)])
    pltpu.sync_copy(bufs[0], acc_sp.at[pl.ds(zb + SLAB - CH, CH)])
    plsc.subcore_barrier()

    pltpu.sync_copy(src_hbm.at[sid, pl.ds(0, NB)], sbufs[0])
    pltpu.sync_copy(dst_hbm.at[cid, sid, pl.ds(0, NB)], dbufs[0])

    def run_group(sbuf, dbuf):
        gd = [pltpu.async_copy(table_sp.at[sbuf.at[j]], bufs[j], gsems[j])
              for j in range(NB)]
        sd = []
        for j in range(NB):
            gd[j].wait()
            sd.append(pltpu.async_copy(bufs[j], acc_sp.at[dbuf.at[j]],
                                       ssem, add=True))
        for j in range(NB):
            sd[j].wait()

    ngroups = nch // NB

    def pair(q, _):
        g1 = 2 * q + 1
        p1 = pltpu.async_copy(src_hbm.at[sid, pl.ds(g1 * NB, NB)],
                              sbufs[1], isem)
        p1b = pltpu.async_copy(dst_hbm.at[cid, sid, pl.ds(g1 * NB, NB)],
                               dbufs[1], isem)
        run_group(sbufs[0], dbufs[0])
        p1.wait()
        p1b.wait()
        nxt = jnp.minimum((g1 + 1) * NB, nch - NB)
        p0 = pltpu.async_copy(src_hbm.at[sid, pl.ds(nxt, NB)],
                              sbufs[0], isem)
        p0b = pltpu.async_copy(dst_hbm.at[cid, sid, pl.ds(nxt, NB)],
                               dbufs[0], isem)
        run_group(sbufs[1], dbufs[1])
        p0.wait()
        p0b.wait()
        return 0
    lax.fori_loop(0, ngroups // 2, pair, 0)
    plsc.subcore_barrier()

    wid = cid * NS + sid
    pltpu.sync_copy(acc_sp.at[pl.ds(sid * SLAB, SLAB)], out_hbm.at[wid])


def _make_sc_scatter(nch, tab_rows):
    mesh = plsc.VectorSubcoreMesh(core_axis_name="c", subcore_axis_name="s")
    return pl.kernel(
        functools.partial(_sc_scatter_body, nch=nch, tab_rows=tab_rows),
        out_type=jax.ShapeDtypeStruct((NW, SLAB, D), jnp.float32),
        mesh=mesh,
        scratch_types=[
            pltpu.VMEM((NB, CH), jnp.int32),
            pltpu.VMEM((NB, CH), jnp.int32),
            pltpu.VMEM((NB, CH), jnp.int32),
            pltpu.VMEM((NB, CH), jnp.int32),
            pltpu.VMEM((CH, D), jnp.float32),
            pltpu.VMEM((CH, D), jnp.float32),
            pltpu.VMEM_SHARED((tab_rows, D), jnp.float32),
            pltpu.VMEM_SHARED((ACC_R, D), jnp.float32),
            pltpu.SemaphoreType.DMA,
            pltpu.SemaphoreType.DMA,
            pltpu.SemaphoreType.DMA,
            pltpu.SemaphoreType.DMA,
        ],
    )


def _h0_body(x_ref, e1_ref, e2_ref, o_ref):
    xb = x_ref[...]
    x0 = xb[:, 0:1]
    x1 = xb[:, 1:2]
    acc = jnp.zeros((BLK, D), jnp.float32)
    for k in range(3):
        acc = acc + jnp.where(x0 == k, 1.0, 0.0) * e1_ref[k:k + 1, :]
        acc = acc + jnp.where(x1 == k, 1.0, 0.0) * e2_ref[k:k + 1, :]
    o_ref[...] = acc


def _layer_body(s_ref, h_ref, c9_ref, e1_ref, e2_ref,
                w1_ref, b1_ref, w2_ref, b2_ref, o_ref, *, last):
    aggr = s_ref[...] + h_ref[...]
    c9 = c9_ref[...]
    for a in range(3):
        for b in range(3):
            cls_cnt = c9[:, 3 * a + b:3 * a + b + 1]
            aggr = aggr + cls_cnt * (e1_ref[a:a + 1, :] + e2_ref[b:b + 1, :])
    aggr = aggr + (e1_ref[4:5, :] + e2_ref[0:1, :])
    hmid = jnp.dot(aggr, w1_ref[...], preferred_element_type=jnp.float32)
    hmid = jnp.maximum(hmid + b1_ref[...], 0.0)
    out = jnp.dot(hmid, w2_ref[...], preferred_element_type=jnp.float32)
    out = out + b2_ref[...]
    if not last:
        out = jnp.maximum(out, 0.0)
    o_ref[...] = out


def _h0_call(xp, e1, e2):
    grid = NPAD // BLK
    return pl.pallas_call(
        _h0_body,
        grid=(grid,),
        in_specs=[
            pl.BlockSpec((BLK, 2), lambda i: (i, 0)),
            pl.BlockSpec((8, D), lambda i: (0, 0)),
            pl.BlockSpec((8, D), lambda i: (0, 0)),
        ],
        out_specs=pl.BlockSpec((BLK, D), lambda i: (i, 0)),
        out_shape=jax.ShapeDtypeStruct((NPAD, D), jnp.float32),
    )(xp, e1, e2)


def _layer_call(s, h, c9, e1l, e2l, w1, b1, w2, b2, last):
    grid = NPAD // BLK
    return pl.pallas_call(
        functools.partial(_layer_body, last=last),
        grid=(grid,),
        in_specs=[
            pl.BlockSpec((BLK, D), lambda i: (i, 0)),
            pl.BlockSpec((BLK, D), lambda i: (i, 0)),
            pl.BlockSpec((BLK, D), lambda i: (i, 0)),
            pl.BlockSpec((8, D), lambda i: (0, 0)),
            pl.BlockSpec((8, D), lambda i: (0, 0)),
            pl.BlockSpec((D, 2 * D), lambda i: (0, 0)),
            pl.BlockSpec((1, 2 * D), lambda i: (0, 0)),
            pl.BlockSpec((2 * D, D), lambda i: (0, 0)),
            pl.BlockSpec((1, D), lambda i: (0, 0)),
        ],
        out_specs=pl.BlockSpec((BLK, D), lambda i: (i, 0)),
        out_shape=jax.ShapeDtypeStruct((NPAD, D), jnp.float32),
    )(s, h, c9, e1l, e2l, w1, b1, w2, b2)


def kernel(x, edge_index, edge_attr, x_emb1, x_emb2, e_emb1, e_emb2,
           W1, b1, W2, b2):
    E = edge_index.shape[1]
    L = W1.shape[0]
    nch = -(-E // (NS * CH))
    nch = -(-nch // (2 * NB)) * (2 * NB)
    ep = NS * nch * CH
    pad = ep - E

    src = edge_index[0].astype(jnp.int32)
    dst = edge_index[1].astype(jnp.int32)
    ar = jnp.arange(E, dtype=jnp.int32)
    dummy = (ar % NS) * SLAB + RSLAB + (ar % (SLAB - RSLAB))
    dum_pad = ((jnp.arange(pad, dtype=jnp.int32) % NS) * SLAB + RSLAB
               + (jnp.arange(pad, dtype=jnp.int32) % (SLAB - RSLAB)))

    def slabify(dl):
        return (dl // RSLAB) * SLAB + (dl % RSLAB)

    dst0 = jnp.where(dst < HALF, slabify(dst), dummy)
    dst1 = jnp.where(dst >= HALF, slabify(dst - HALF), dummy)
    dst_p = jnp.stack([
        jnp.concatenate([dst0, dum_pad]),
        jnp.concatenate([dst1, dum_pad]),
    ]).reshape(NC, NS, nch, CH)
    src_p = jnp.concatenate([src, jnp.zeros((pad,), jnp.int32)]
                            ).reshape(NS, nch, CH)
    cls = ((edge_attr[:, 0] * 3 + edge_attr[:, 1]).astype(jnp.int32)
           + 16 * (ar % REPC))
    cls_p = jnp.concatenate(
        [cls, 15 + 16 * (jnp.arange(pad, dtype=jnp.int32) % REPC)]
    ).reshape(NS, nch, CH)

    xp = jnp.pad(x.astype(jnp.int32), ((0, NPAD - N), (0, 0)))
    eye16 = jnp.tile(jnp.pad(jnp.eye(16, dtype=jnp.float32),
                             ((0, 0), (0, D - 16))), (REPC, 1))
    e1pad = jnp.pad(x_emb1[:3], ((0, 5), (0, 0)))
    e2pad = jnp.pad(x_emb2[:3], ((0, 5), (0, 0)))

    sc_aggr = _make_sc_scatter(nch, NPAD)
    sc_c9 = _make_sc_scatter(nch, 16 * REPC)

    h = _h0_call(xp, e1pad, e2pad)
    c9 = sc_c9(eye16, cls_p, dst_p)

    def unslab(p):
        return p[:, :RSLAB, :].reshape(NPAD, D)

    c9 = unslab(c9)
    for l in range(L):
        s = unslab(sc_aggr(h, src_p, dst_p))
        e1l = jnp.pad(e_emb1[l], ((0, 8 - e_emb1.shape[1]), (0, 0)))
        e2l = jnp.pad(e_emb2[l], ((0, 8 - e_emb2.shape[1]), (0, 0)))
        h = _layer_call(s, h, c9, e1l, e2l,
                        W1[l], b1[l].reshape(1, -1),
                        W2[l], b2[l].reshape(1, -1), last=(l == L - 1))
    return h[:N]


# speedup vs baseline: 1.2783x; 1.2783x over previous
import functools

import jax
import jax.numpy as jnp
from jax import lax
from jax.experimental import pallas as pl
from jax.experimental.pallas import tpu as pltpu
from jax.experimental.pallas import tpu_sc as plsc

N = 10000
D = 128
NPAD = 10112
HALF = NPAD // 2
RSLAB = 316
SLAB = 320
NC = 2
NS = 16
NW = NC * NS
CH = 16
GRP = 16
BLK = 632
ACC_R = NS * SLAB
REPC = 64


def _zero_rows(buf, nrows, width):
    def body(r, _):
        for j in range(width // 16):
            buf[r, pl.ds(j * 16, 16)] = jnp.zeros((16,), jnp.float32)
        return 0
    lax.fori_loop(0, nrows, body, 0)


def _sc_scatter_body(table_hbm, src_hbm, dst_hbm, out_hbm,
                     sb0, sb1, db0, db1, b0, b1, table_sp, acc_sp,
                     gs0, gs1, ss0, ss1, isem, nch, tab_rows):
    bufs = (b0, b1)
    gsems = (gs0, gs1)
    ssems = (ss0, ss1)
    sbufs = (sb0, sb1)
    dbufs = (db0, db1)

    cid = lax.axis_index("c")
    sid = lax.axis_index("s")

    tpt = tab_rows // NS
    pltpu.sync_copy(table_hbm.at[pl.ds(sid * tpt, tpt)],
                    table_sp.at[pl.ds(sid * tpt, tpt)])

    _zero_rows(bufs[0], CH, D)
    zb = sid * SLAB
    for k in range(SLAB // CH):
        pltpu.sync_copy(bufs[0], acc_sp.at[pl.ds(zb + k * CH, CH)])
    pltpu.sync_copy(bufs[0], acc_sp.at[pl.ds(zb + SLAB - CH, CH)])
    plsc.subcore_barrier()

    HG = GRP // 2

    def srcref(k):
        sb = sbufs[0] if k < HG else sbufs[1]
        return sb.at[pl.ds((k % HG) * CH, CH)]

    def dstref(k):
        db = dbufs[0] if k < HG else dbufs[1]
        return db.at[k % HG]

    def gwait(k):
        j = k % 2
        pltpu.make_async_copy(table_sp.at[srcref(k % GRP)], bufs[j],
                              gsems[j]).wait()

    def swait(k):
        j = k % 2
        pltpu.make_async_copy(bufs[j], acc_sp.at[dstref(k % GRP)],
                              ssems[j]).wait()

    pltpu.sync_copy(src_hbm.at[sid, 0, pl.ds(0, HG * CH)], sbufs[0])
    pltpu.sync_copy(dst_hbm.at[cid, sid, pl.ds(0, HG)], dbufs[0])
    pltpu.async_copy(table_sp.at[srcref(0)], bufs[0], gsems[0])

    def iteration(p, _):
        base = p * GRP
        pf = []
        for k in range(GRP):
            j = k % 2
            gwait(k)
            pltpu.async_copy(bufs[j], acc_sp.at[dstref(k)], ssems[j],
                             add=True)
            if k == 0:
                @pl.when(p > 0)
                def _():
                    swait(k - 1)
            else:
                swait(k - 1)
            if k == 0:
                pf = [pltpu.async_copy(
                          src_hbm.at[sid, 0, pl.ds((base + HG) * CH, HG * CH)],
                          sbufs[1], isem),
                      pltpu.async_copy(
                          dst_hbm.at[cid, sid, pl.ds(base + HG, HG)],
                          dbufs[1], isem)]
            if k == HG:
                nxt = jnp.minimum(base + GRP, nch - HG)
                pf = [pltpu.async_copy(
                          src_hbm.at[sid, 0, pl.ds(nxt * CH, HG * CH)],
                          sbufs[0], isem),
                      pltpu.async_copy(
                          dst_hbm.at[cid, sid, pl.ds(nxt, HG)],
                          dbufs[0], isem)]
            if k in (HG - 1, GRP - 1):
                for d in pf:
                    d.wait()
            pltpu.async_copy(table_sp.at[srcref((k + 1) % GRP)],
                             bufs[1 - j], gsems[1 - j])
        return 0
    lax.fori_loop(0, nch // GRP, iteration, 0)
    gwait(0)
    swait(GRP - 1)
    plsc.subcore_barrier()

    wid = cid * NS + sid
    pltpu.sync_copy(acc_sp.at[pl.ds(sid * SLAB, SLAB)], out_hbm.at[wid])


def _make_sc_scatter(nch, tab_rows):
    mesh = plsc.VectorSubcoreMesh(core_axis_name="c", subcore_axis_name="s")
    return pl.kernel(
        functools.partial(_sc_scatter_body, nch=nch, tab_rows=tab_rows),
        out_type=jax.ShapeDtypeStruct((NW, SLAB, D), jnp.float32),
        mesh=mesh,
        scratch_types=[
            pltpu.VMEM((GRP // 2 * CH,), jnp.int32),
            pltpu.VMEM((GRP // 2 * CH,), jnp.int32),
            pltpu.VMEM((GRP // 2, CH), jnp.int32),
            pltpu.VMEM((GRP // 2, CH), jnp.int32),
            pltpu.VMEM((CH, D), jnp.float32),
            pltpu.VMEM((CH, D), jnp.float32),
            pltpu.VMEM_SHARED((tab_rows, D), jnp.float32),
            pltpu.VMEM_SHARED((ACC_R, D), jnp.float32),
            pltpu.SemaphoreType.DMA,
            pltpu.SemaphoreType.DMA,
            pltpu.SemaphoreType.DMA,
            pltpu.SemaphoreType.DMA,
            pltpu.SemaphoreType.DMA,
        ],
    )


def _h0_body(x_ref, e1_ref, e2_ref, o_ref):
    xb = x_ref[...]
    x0 = xb[:, 0:1]
    x1 = xb[:, 1:2]
    acc = jnp.zeros((BLK, D), jnp.float32)
    for k in range(3):
        acc = acc + jnp.where(x0 == k, 1.0, 0.0) * e1_ref[k:k + 1, :]
        acc = acc + jnp.where(x1 == k, 1.0, 0.0) * e2_ref[k:k + 1, :]
    o_ref[...] = acc


def _layer_body(s_ref, h_ref, c9_ref, e1_ref, e2_ref,
                w1_ref, b1_ref, w2_ref, b2_ref, o_ref, *, last):
    aggr = s_ref[...] + h_ref[...]
    c9 = c9_ref[...]
    for a in range(3):
        for b in range(3):
            cls_cnt = c9[:, 3 * a + b:3 * a + b + 1]
            aggr = aggr + cls_cnt * (e1_ref[a:a + 1, :] + e2_ref[b:b + 1, :])
    aggr = aggr + (e1_ref[4:5, :] + e2_ref[0:1, :])
    hmid = jnp.dot(aggr, w1_ref[...], preferred_element_type=jnp.float32)
    hmid = jnp.maximum(hmid + b1_ref[...], 0.0)
    out = jnp.dot(hmid, w2_ref[...], preferred_element_type=jnp.float32)
    out = out + b2_ref[...]
    if not last:
        out = jnp.maximum(out, 0.0)
    o_ref[...] = out


def _h0_call(xp, e1, e2):
    grid = NPAD // BLK
    return pl.pallas_call(
        _h0_body,
        grid=(grid,),
        in_specs=[
            pl.BlockSpec((BLK, 2), lambda i: (i, 0)),
            pl.BlockSpec((8, D), lambda i: (0, 0)),
            pl.BlockSpec((8, D), lambda i: (0, 0)),
        ],
        out_specs=pl.BlockSpec((BLK, D), lambda i: (i, 0)),
        out_shape=jax.ShapeDtypeStruct((NPAD, D), jnp.float32),
    )(xp, e1, e2)


def _layer_call(s, h, c9, e1l, e2l, w1, b1, w2, b2, last):
    grid = NPAD // BLK
    return pl.pallas_call(
        functools.partial(_layer_body, last=last),
        grid=(grid,),
        in_specs=[
            pl.BlockSpec((BLK, D), lambda i: (i, 0)),
            pl.BlockSpec((BLK, D), lambda i: (i, 0)),
            pl.BlockSpec((BLK, D), lambda i: (i, 0)),
            pl.BlockSpec((8, D), lambda i: (0, 0)),
            pl.BlockSpec((8, D), lambda i: (0, 0)),
            pl.BlockSpec((D, 2 * D), lambda i: (0, 0)),
            pl.BlockSpec((1, 2 * D), lambda i: (0, 0)),
            pl.BlockSpec((2 * D, D), lambda i: (0, 0)),
            pl.BlockSpec((1, D), lambda i: (0, 0)),
        ],
        out_specs=pl.BlockSpec((BLK, D), lambda i: (i, 0)),
        out_shape=jax.ShapeDtypeStruct((NPAD, D), jnp.float32),
    )(s, h, c9, e1l, e2l, w1, b1, w2, b2)


def kernel(x, edge_index, edge_attr, x_emb1, x_emb2, e_emb1, e_emb2,
           W1, b1, W2, b2):
    E = edge_index.shape[1]
    L = W1.shape[0]
    nch = -(-E // (NS * CH))
    nch = -(-nch // GRP) * GRP
    ep = NS * nch * CH
    pad = ep - E

    src = edge_index[0].astype(jnp.int32)
    dst = edge_index[1].astype(jnp.int32)
    ar = jnp.arange(E, dtype=jnp.int32)
    dummy = (ar % NS) * SLAB + RSLAB + (ar % (SLAB - RSLAB))
    dum_pad = ((jnp.arange(pad, dtype=jnp.int32) % NS) * SLAB + RSLAB
               + (jnp.arange(pad, dtype=jnp.int32) % (SLAB - RSLAB)))

    def slabify(dl):
        return (dl // RSLAB) * SLAB + (dl % RSLAB)

    dst0 = jnp.where(dst < HALF, slabify(dst), dummy)
    dst1 = jnp.where(dst >= HALF, slabify(dst - HALF), dummy)
    dst_p = jnp.stack([
        jnp.concatenate([dst0, dum_pad]),
        jnp.concatenate([dst1, dum_pad]),
    ]).reshape(NC, NS, nch, CH)
    src_p = jnp.concatenate([src, jnp.zeros((pad,), jnp.int32)]
                            ).reshape(NS, 1, nch * CH)
    cls = ((edge_attr[:, 0] * 3 + edge_attr[:, 1]).astype(jnp.int32)
           + 16 * (ar % REPC))
    cls_p = jnp.concatenate(
        [cls, 15 + 16 * (jnp.arange(pad, dtype=jnp.int32) % REPC)]
    ).reshape(NS, 1, nch * CH)

    xp = jnp.pad(x.astype(jnp.int32), ((0, NPAD - N), (0, 0)))
    eye16 = jnp.tile(jnp.pad(jnp.eye(16, dtype=jnp.float32),
                             ((0, 0), (0, D - 16))), (REPC, 1))
    e1pad = jnp.pad(x_emb1[:3], ((0, 5), (0, 0)))
    e2pad = jnp.pad(x_emb2[:3], ((0, 5), (0, 0)))

    sc_aggr = _make_sc_scatter(nch, NPAD)
    sc_c9 = _make_sc_scatter(nch, 16 * REPC)

    h = _h0_call(xp, e1pad, e2pad)
    c9 = sc_c9(eye16, cls_p, dst_p)

    def unslab(p):
        return p[:, :RSLAB, :].reshape(NPAD, D)

    c9 = unslab(c9)
    for l in range(L):
        s = unslab(sc_aggr(h, src_p, dst_p))
        e1l = jnp.pad(e_emb1[l], ((0, 8 - e_emb1.shape[1]), (0, 0)))
        e2l = jnp.pad(e_emb2[l], ((0, 8 - e_emb2.shape[1]), (0, 0)))
        h = _layer_call(s, h, c9, e1l, e2l,
                        W1[l], b1[l].reshape(1, -1),
                        W2[l], b2[l].reshape(1, -1), last=(l == L - 1))
    return h[:N]


# speedup vs baseline: 1.3962x; 1.0922x over previous
import functools

import jax
import jax.numpy as jnp
from jax import lax
from jax.experimental import pallas as pl
from jax.experimental.pallas import tpu as pltpu
from jax.experimental.pallas import tpu_sc as plsc

N = 10000
D = 128
NPAD = 10112
HALF = NPAD // 2
RSLAB = 316
SLAB = 320
NC = 2
NS = 16
NW = NC * NS
CH = 32
GRP = 8
TROWS = 10048
BLK = 632
ACC_R = NS * SLAB
REPC = 64


def _zero_rows(buf, nrows, width):
    def body(r, _):
        for j in range(width // 16):
            buf[r, pl.ds(j * 16, 16)] = jnp.zeros((16,), jnp.float32)
        return 0
    lax.fori_loop(0, nrows, body, 0)


def _sc_scatter_body(table_hbm, src_hbm, dst_hbm, out_hbm,
                     sb0, sb1, db0, db1, b0, b1, table_sp, acc_sp,
                     gs0, gs1, ss0, ss1, isem, nch, tab_rows):
    bufs = (b0, b1)
    gsems = (gs0, gs1)
    ssems = (ss0, ss1)
    sbufs = (sb0, sb1)
    dbufs = (db0, db1)

    cid = lax.axis_index("c")
    sid = lax.axis_index("s")

    if tab_rows % (NS * 8) == 0:
        tpt = tab_rows // NS
        pltpu.sync_copy(table_hbm.at[pl.ds(sid * tpt, tpt)],
                        table_sp.at[pl.ds(sid * tpt, tpt)])
    else:
        tpt = -(-(tab_rows // NS) // 8) * 8
        last = tab_rows - (NS - 1) * tpt

        @pl.when(sid < NS - 1)
        def _():
            pltpu.sync_copy(table_hbm.at[pl.ds(sid * tpt, tpt)],
                            table_sp.at[pl.ds(sid * tpt, tpt)])

        @pl.when(sid == NS - 1)
        def _():
            pltpu.sync_copy(table_hbm.at[pl.ds((NS - 1) * tpt, last)],
                            table_sp.at[pl.ds((NS - 1) * tpt, last)])

    _zero_rows(bufs[0], CH, D)
    zb = sid * SLAB
    for k in range(SLAB // CH):
        pltpu.sync_copy(bufs[0], acc_sp.at[pl.ds(zb + k * CH, CH)])
    pltpu.sync_copy(bufs[0], acc_sp.at[pl.ds(zb + SLAB - CH, CH)])
    plsc.subcore_barrier()

    HG = GRP // 2

    def srcref(k):
        sb = sbufs[0] if k < HG else sbufs[1]
        return sb.at[pl.ds((k % HG) * CH, CH)]

    def dstref(k):
        db = dbufs[0] if k < HG else dbufs[1]
        return db.at[k % HG]

    def gwait(k):
        j = k % 2
        pltpu.make_async_copy(table_sp.at[srcref(k % GRP)], bufs[j],
                              gsems[j]).wait()

    def swait(k):
        j = k % 2
        pltpu.make_async_copy(bufs[j], acc_sp.at[dstref(k % GRP)],
                              ssems[j]).wait()

    pltpu.sync_copy(src_hbm.at[sid, 0, pl.ds(0, HG * CH)], sbufs[0])
    pltpu.sync_copy(dst_hbm.at[cid, sid, pl.ds(0, HG)], dbufs[0])
    pltpu.async_copy(table_sp.at[srcref(0)], bufs[0], gsems[0])

    def iteration(p, _):
        base = p * GRP
        pf = []
        for k in range(GRP):
            j = k % 2
            gwait(k)
            pltpu.async_copy(bufs[j], acc_sp.at[dstref(k)], ssems[j],
                             add=True)
            if k == 0:
                @pl.when(p > 0)
                def _():
                    swait(k - 1)
            else:
                swait(k - 1)
            if k == 0:
                pf = [pltpu.async_copy(
                          src_hbm.at[sid, 0, pl.ds((base + HG) * CH, HG * CH)],
                          sbufs[1], isem),
                      pltpu.async_copy(
                          dst_hbm.at[cid, sid, pl.ds(base + HG, HG)],
                          dbufs[1], isem)]
            if k == HG:
                nxt = jnp.minimum(base + GRP, nch - HG)
                pf = [pltpu.async_copy(
                          src_hbm.at[sid, 0, pl.ds(nxt * CH, HG * CH)],
                          sbufs[0], isem),
                      pltpu.async_copy(
                          dst_hbm.at[cid, sid, pl.ds(nxt, HG)],
                          dbufs[0], isem)]
            if k in (HG - 1, GRP - 1):
                for d in pf:
                    d.wait()
            pltpu.async_copy(table_sp.at[srcref((k + 1) % GRP)],
                             bufs[1 - j], gsems[1 - j])
        return 0
    lax.fori_loop(0, nch // GRP, iteration, 0)
    gwait(0)
    swait(GRP - 1)
    plsc.subcore_barrier()

    wid = cid * NS + sid
    pltpu.sync_copy(acc_sp.at[pl.ds(sid * SLAB, SLAB)], out_hbm.at[wid])


def _make_sc_scatter(nch, tab_rows):
    mesh = plsc.VectorSubcoreMesh(core_axis_name="c", subcore_axis_name="s")
    return pl.kernel(
        functools.partial(_sc_scatter_body, nch=nch, tab_rows=tab_rows),
        out_type=jax.ShapeDtypeStruct((NW, SLAB, D), jnp.float32),
        mesh=mesh,
        scratch_types=[
            pltpu.VMEM((GRP // 2 * CH,), jnp.int32),
            pltpu.VMEM((GRP // 2 * CH,), jnp.int32),
            pltpu.VMEM((GRP // 2, CH), jnp.int32),
            pltpu.VMEM((GRP // 2, CH), jnp.int32),
            pltpu.VMEM((CH, D), jnp.float32),
            pltpu.VMEM((CH, D), jnp.float32),
            pltpu.VMEM_SHARED((tab_rows, D), jnp.float32),
            pltpu.VMEM_SHARED((ACC_R, D), jnp.float32),
            pltpu.SemaphoreType.DMA,
            pltpu.SemaphoreType.DMA,
            pltpu.SemaphoreType.DMA,
            pltpu.SemaphoreType.DMA,
            pltpu.SemaphoreType.DMA,
        ],
    )


def _h0_body(x_ref, e1_ref, e2_ref, o_ref):
    xb = x_ref[...]
    x0 = xb[:, 0:1]
    x1 = xb[:, 1:2]
    acc = jnp.zeros((BLK, D), jnp.float32)
    for k in range(3):
        acc = acc + jnp.where(x0 == k, 1.0, 0.0) * e1_ref[k:k + 1, :]
        acc = acc + jnp.where(x1 == k, 1.0, 0.0) * e2_ref[k:k + 1, :]
    o_ref[...] = acc


def _layer_body(s_ref, h_ref, c9_ref, e1_ref, e2_ref,
                w1_ref, b1_ref, w2_ref, b2_ref, o_ref, *, last):
    aggr = s_ref[...] + h_ref[...]
    c9 = c9_ref[...]
    for a in range(3):
        for b in range(3):
            cls_cnt = c9[:, 3 * a + b:3 * a + b + 1]
            aggr = aggr + cls_cnt * (e1_ref[a:a + 1, :] + e2_ref[b:b + 1, :])
    aggr = aggr + (e1_ref[4:5, :] + e2_ref[0:1, :])
    hmid = jnp.dot(aggr, w1_ref[...], preferred_element_type=jnp.float32)
    hmid = jnp.maximum(hmid + b1_ref[...], 0.0)
    out = jnp.dot(hmid, w2_ref[...], preferred_element_type=jnp.float32)
    out = out + b2_ref[...]
    if not last:
        out = jnp.maximum(out, 0.0)
    o_ref[...] = out


def _h0_call(xp, e1, e2):
    grid = NPAD // BLK
    return pl.pallas_call(
        _h0_body,
        grid=(grid,),
        in_specs=[
            pl.BlockSpec((BLK, 2), lambda i: (i, 0)),
            pl.BlockSpec((8, D), lambda i: (0, 0)),
            pl.BlockSpec((8, D), lambda i: (0, 0)),
        ],
        out_specs=pl.BlockSpec((BLK, D), lambda i: (i, 0)),
        out_shape=jax.ShapeDtypeStruct((NPAD, D), jnp.float32),
    )(xp, e1, e2)


def _layer_call(s, h, c9, e1l, e2l, w1, b1, w2, b2, last):
    grid = NPAD // BLK
    return pl.pallas_call(
        functools.partial(_layer_body, last=last),
        grid=(grid,),
        in_specs=[
            pl.BlockSpec((BLK, D), lambda i: (i, 0)),
            pl.BlockSpec((BLK, D), lambda i: (i, 0)),
            pl.BlockSpec((BLK, D), lambda i: (i, 0)),
            pl.BlockSpec((8, D), lambda i: (0, 0)),
            pl.BlockSpec((8, D), lambda i: (0, 0)),
            pl.BlockSpec((D, 2 * D), lambda i: (0, 0)),
            pl.BlockSpec((1, 2 * D), lambda i: (0, 0)),
            pl.BlockSpec((2 * D, D), lambda i: (0, 0)),
            pl.BlockSpec((1, D), lambda i: (0, 0)),
        ],
        out_specs=pl.BlockSpec((BLK, D), lambda i: (i, 0)),
        out_shape=jax.ShapeDtypeStruct((NPAD, D), jnp.float32),
    )(s, h, c9, e1l, e2l, w1, b1, w2, b2)


def kernel(x, edge_index, edge_attr, x_emb1, x_emb2, e_emb1, e_emb2,
           W1, b1, W2, b2):
    E = edge_index.shape[1]
    L = W1.shape[0]
    nch = -(-E // (NS * CH))
    nch = -(-nch // GRP) * GRP
    ep = NS * nch * CH
    pad = ep - E

    src = edge_index[0].astype(jnp.int32)
    dst = edge_index[1].astype(jnp.int32)
    ar = jnp.arange(E, dtype=jnp.int32)
    dummy = (ar % NS) * SLAB + RSLAB + (ar % (SLAB - RSLAB))
    dum_pad = ((jnp.arange(pad, dtype=jnp.int32) % NS) * SLAB + RSLAB
               + (jnp.arange(pad, dtype=jnp.int32) % (SLAB - RSLAB)))

    def slabify(dl):
        return (dl // RSLAB) * SLAB + (dl % RSLAB)

    dst0 = jnp.where(dst < HALF, slabify(dst), dummy)
    dst1 = jnp.where(dst >= HALF, slabify(dst - HALF), dummy)
    dst_p = jnp.stack([
        jnp.concatenate([dst0, dum_pad]),
        jnp.concatenate([dst1, dum_pad]),
    ]).reshape(NC, NS, nch, CH)
    src_p = jnp.concatenate([src, jnp.zeros((pad,), jnp.int32)]
                            ).reshape(NS, 1, nch * CH)
    cls = ((edge_attr[:, 0] * 3 + edge_attr[:, 1]).astype(jnp.int32)
           + 16 * (ar % REPC))
    cls_p = jnp.concatenate(
        [cls, 15 + 16 * (jnp.arange(pad, dtype=jnp.int32) % REPC)]
    ).reshape(NS, 1, nch * CH)

    xp = jnp.pad(x.astype(jnp.int32), ((0, NPAD - N), (0, 0)))
    eye16 = jnp.tile(jnp.pad(jnp.eye(16, dtype=jnp.float32),
                             ((0, 0), (0, D - 16))), (REPC, 1))
    e1pad = jnp.pad(x_emb1[:3], ((0, 5), (0, 0)))
    e2pad = jnp.pad(x_emb2[:3], ((0, 5), (0, 0)))

    sc_aggr = _make_sc_scatter(nch, TROWS)
    sc_c9 = _make_sc_scatter(nch, 16 * REPC)

    h = _h0_call(xp, e1pad, e2pad)
    c9 = sc_c9(eye16, cls_p, dst_p)

    def unslab(p):
        return p[:, :RSLAB, :].reshape(NPAD, D)

    c9 = unslab(c9)
    for l in range(L):
        s = unslab(sc_aggr(h, src_p, dst_p))
        e1l = jnp.pad(e_emb1[l], ((0, 8 - e_emb1.shape[1]), (0, 0)))
        e2l = jnp.pad(e_emb2[l], ((0, 8 - e_emb2.shape[1]), (0, 0)))
        h = _layer_call(s, h, c9, e1l, e2l,
                        W1[l], b1[l].reshape(1, -1),
                        W2[l], b2[l].reshape(1, -1), last=(l == L - 1))
    return h[:N]
